# baseline (speedup 1.0000x reference)
import jax
import jax.numpy as jnp
from jax.experimental import pallas as pl

_B = 4
_S = 8193
_D = 2048
_BLOCK = 896
_NB = (_S + _BLOCK - 1) // _BLOCK


def _onehot_body(pt_ref, out_ref):
    ids4 = pt_ref[...]
    ids8 = jnp.tile(ids4, (1, 2))
    idsb = jnp.broadcast_to(ids8[:, :, None], (_BLOCK, 8, 128))
    lane = jax.lax.broadcasted_iota(jnp.int32, (8, 8, 128), 2)
    kidx = jax.lax.broadcasted_iota(jnp.int32, (8, 8, 128), 0)
    rr = jax.lax.broadcasted_iota(jnp.int32, (8, 8, 128), 1)
    voff = lane + (((kidx << 1) + (rr >> 2)) << 7)
    out4 = (idsb[:, None, :, :] == voff[None]).astype(jnp.float32)
    out_ref[...] = out4.reshape(_BLOCK, 64, 128)


def kernel(input_ids):
    padded = jnp.pad(input_ids, ((0, 0), (1, 0)), constant_values=0)
    pt = padded.T
    out3 = pl.pallas_call(
        _onehot_body,
        grid=(_NB,),
        in_specs=[pl.BlockSpec((_BLOCK, 4), lambda i: (i, 0))],
        out_specs=pl.BlockSpec((_BLOCK, 64, 128), lambda i: (i, 0, 0)),
        out_shape=jax.ShapeDtypeStruct((_S, 64, 128), jnp.float32),
    )(pt)
    out = out3.reshape(_S, 16, _B, 128).transpose(2, 0, 1, 3)
    return out.reshape(_B, _S, _D)


# speedup vs baseline: 1.0072x; 1.0072x over previous
import jax
import jax.numpy as jnp
from jax.experimental import pallas as pl

_B = 4
_S = 8193
_D = 2048
_BLOCK = 640
_NB = (_S + _BLOCK - 1) // _BLOCK


def _onehot_body(pt_ref, out_ref):
    ids4 = pt_ref[...]
    ids8 = jnp.tile(ids4, (1, 2))
    idsb = jnp.broadcast_to(ids8[:, :, None], (_BLOCK, 8, 128))
    lane = jax.lax.broadcasted_iota(jnp.int32, (8, 8, 128), 2)
    kidx = jax.lax.broadcasted_iota(jnp.int32, (8, 8, 128), 0)
    rr = jax.lax.broadcasted_iota(jnp.int32, (8, 8, 128), 1)
    voff = lane + (((kidx << 1) + (rr >> 2)) << 7)
    out4 = (idsb[:, None, :, :] == voff[None]).astype(jnp.float32)
    out_ref[...] = out4.reshape(_BLOCK, 64, 128)


def kernel(input_ids):
    padded = jnp.pad(input_ids, ((0, 0), (1, 0)), constant_values=0)
    pt = padded.T
    out3 = pl.pallas_call(
        _onehot_body,
        grid=(_NB,),
        in_specs=[pl.BlockSpec((_BLOCK, 4), lambda i: (i, 0))],
        out_specs=pl.BlockSpec((_BLOCK, 64, 128), lambda i: (i, 0, 0)),
        out_shape=jax.ShapeDtypeStruct((_S, 64, 128), jnp.float32),
    )(pt)
    out = out3.reshape(_S, 16, _B, 128).transpose(2, 0, 1, 3)
    return out.reshape(_B, _S, _D)


# speedup vs baseline: 1.0231x; 1.0158x over previous
import jax
import jax.numpy as jnp
from jax.experimental import pallas as pl

_B = 4
_S = 8193
_D = 2048
_BLOCK = 384
_NB = (_S + _BLOCK - 1) // _BLOCK


def _onehot_body(pt_ref, out_ref):
    ids4 = pt_ref[...]
    ids8 = jnp.tile(ids4, (1, 2))
    idsb = jnp.broadcast_to(ids8[:, :, None], (_BLOCK, 8, 128))
    lane = jax.lax.broadcasted_iota(jnp.int32, (8, 8, 128), 2)
    kidx = jax.lax.broadcasted_iota(jnp.int32, (8, 8, 128), 0)
    rr = jax.lax.broadcasted_iota(jnp.int32, (8, 8, 128), 1)
    voff = lane + (((kidx << 1) + (rr >> 2)) << 7)
    out4 = (idsb[:, None, :, :] == voff[None]).astype(jnp.float32)
    out_ref[...] = out4.reshape(_BLOCK, 64, 128)


def kernel(input_ids):
    padded = jnp.pad(input_ids, ((0, 0), (1, 0)), constant_values=0)
    pt = padded.T
    out3 = pl.pallas_call(
        _onehot_body,
        grid=(_NB,),
        in_specs=[pl.BlockSpec((_BLOCK, 4), lambda i: (i, 0))],
        out_specs=pl.BlockSpec((_BLOCK, 64, 128), lambda i: (i, 0, 0)),
        out_shape=jax.ShapeDtypeStruct((_S, 64, 128), jnp.float32),
    )(pt)
    out = out3.reshape(_S, 16, _B, 128).transpose(2, 0, 1, 3)
    return out.reshape(_B, _S, _D)
